# baseline (device time: 15317 ns/iter reference)
import jax
import jax.numpy as jnp
from jax import lax
from jax.experimental import pallas as pl
from jax.experimental.pallas import tpu as pltpu

M = 512
N_OUT = 512
MQ = 256
C = 8
R = MQ // C


def kernel(x):
    def body(x_ref, out_ref, send_buf, recv_y_buf, recv_x_buf,
             y_send_sems, y_recv_sems, x_send_sems, x_recv_sems):
        my_x = lax.axis_index("x")
        my_y = lax.axis_index("y")
        other_x = 1 - my_x
        other_y = 1 - my_y

        barrier_sem = pltpu.get_barrier_semaphore()
        pl.semaphore_signal(barrier_sem, inc=1, device_id=(my_x, other_y),
                            device_id_type=pl.DeviceIdType.MESH)
        pl.semaphore_signal(barrier_sem, inc=1, device_id=(other_x, my_y),
                            device_id_type=pl.DeviceIdType.MESH)
        pl.semaphore_wait(barrier_sem, 2)

        send_buf[:, :] = x_ref[0, pl.ds(my_x * MQ, MQ),
                               pl.ds(other_y * N_OUT, N_OUT)]
        y_rdmas = []
        for c in range(C):
            rdma = pltpu.make_async_remote_copy(
                src_ref=send_buf.at[pl.ds(c * R, R)],
                dst_ref=recv_y_buf.at[pl.ds(c * R, R)],
                send_sem=y_send_sems.at[c],
                recv_sem=y_recv_sems.at[c],
                device_id=(my_x, other_y),
                device_id_type=pl.DeviceIdType.MESH,
            )
            rdma.start()
            y_rdmas.append(rdma)

        x_rdmas = []
        for c in range(C):
            y_rdmas[c].wait_recv()
            rdma = pltpu.make_async_remote_copy(
                src_ref=recv_y_buf.at[pl.ds(c * R, R)],
                dst_ref=recv_x_buf.at[pl.ds(c * R, R)],
                send_sem=x_send_sems.at[c],
                recv_sem=x_recv_sems.at[c],
                device_id=(other_x, my_y),
                device_id_type=pl.DeviceIdType.MESH,
            )
            rdma.start()
            x_rdmas.append(rdma)

        out_ref[pl.ds(my_x * MQ, MQ), :] = (
            x_ref[0, pl.ds(my_x * MQ, MQ), pl.ds(my_y * N_OUT, N_OUT)]
            + recv_y_buf[:, :]
        )

        for c in range(C):
            x_rdmas[c].wait_recv()
        out_ref[pl.ds(other_x * MQ, MQ), :] = (
            x_ref[0, pl.ds(other_x * MQ, MQ), pl.ds(my_y * N_OUT, N_OUT)]
            + recv_x_buf[:, :]
        )

        for c in range(C):
            y_rdmas[c].wait_send()
            x_rdmas[c].wait_send()

    return pl.pallas_call(
        body,
        out_shape=jax.ShapeDtypeStruct((M, N_OUT), jnp.float32),
        in_specs=[pl.BlockSpec(memory_space=pltpu.VMEM)],
        out_specs=pl.BlockSpec(memory_space=pltpu.VMEM),
        scratch_shapes=[
            pltpu.VMEM((MQ, N_OUT), jnp.float32),
            pltpu.VMEM((MQ, N_OUT), jnp.float32),
            pltpu.VMEM((MQ, N_OUT), jnp.float32),
            pltpu.SemaphoreType.DMA((C,)),
            pltpu.SemaphoreType.DMA((C,)),
            pltpu.SemaphoreType.DMA((C,)),
            pltpu.SemaphoreType.DMA((C,)),
        ],
        compiler_params=pltpu.CompilerParams(collective_id=0),
    )(x)


# device time: 15214 ns/iter; 1.0068x vs baseline; 1.0068x over previous
import jax
import jax.numpy as jnp
from jax import lax
from jax.experimental import pallas as pl
from jax.experimental.pallas import tpu as pltpu

M = 512
N_OUT = 512
MQ = 256
C = 16
R = MQ // C


def kernel(x):
    def body(x_ref, out_ref, recv_y_buf, recv_x_buf,
             y_send_sems, y_recv_sems, x_send_sems, x_recv_sems):
        my_x = lax.axis_index("x")
        my_y = lax.axis_index("y")
        other_x = 1 - my_x
        other_y = 1 - my_y

        barrier_sem = pltpu.get_barrier_semaphore()
        pl.semaphore_signal(barrier_sem, inc=1, device_id=(my_x, other_y),
                            device_id_type=pl.DeviceIdType.MESH)
        pl.semaphore_signal(barrier_sem, inc=1, device_id=(other_x, my_y),
                            device_id_type=pl.DeviceIdType.MESH)
        pl.semaphore_wait(barrier_sem, 2)

        y_rdmas = []
        for c in range(C):
            rdma = pltpu.make_async_remote_copy(
                src_ref=x_ref.at[0, pl.ds(my_x * MQ + c * R, R),
                                 pl.ds(other_y * N_OUT, N_OUT)],
                dst_ref=recv_y_buf.at[pl.ds(c * R, R)],
                send_sem=y_send_sems.at[c],
                recv_sem=y_recv_sems.at[c],
                device_id=(my_x, other_y),
                device_id_type=pl.DeviceIdType.MESH,
            )
            rdma.start()
            y_rdmas.append(rdma)

        x_rdmas = []
        for c in range(C):
            y_rdmas[c].wait_recv()
            rdma = pltpu.make_async_remote_copy(
                src_ref=recv_y_buf.at[pl.ds(c * R, R)],
                dst_ref=recv_x_buf.at[pl.ds(c * R, R)],
                send_sem=x_send_sems.at[c],
                recv_sem=x_recv_sems.at[c],
                device_id=(other_x, my_y),
                device_id_type=pl.DeviceIdType.MESH,
            )
            rdma.start()
            x_rdmas.append(rdma)
            out_ref[pl.ds(my_x * MQ + c * R, R), :] = (
                x_ref[0, pl.ds(my_x * MQ + c * R, R),
                      pl.ds(my_y * N_OUT, N_OUT)]
                + recv_y_buf[pl.ds(c * R, R), :]
            )

        for c in range(C):
            x_rdmas[c].wait_recv()
            out_ref[pl.ds(other_x * MQ + c * R, R), :] = (
                x_ref[0, pl.ds(other_x * MQ + c * R, R),
                      pl.ds(my_y * N_OUT, N_OUT)]
                + recv_x_buf[pl.ds(c * R, R), :]
            )

        for c in range(C):
            y_rdmas[c].wait_send()
            x_rdmas[c].wait_send()

    return pl.pallas_call(
        body,
        out_shape=jax.ShapeDtypeStruct((M, N_OUT), jnp.float32),
        in_specs=[pl.BlockSpec(memory_space=pltpu.VMEM)],
        out_specs=pl.BlockSpec(memory_space=pltpu.VMEM),
        scratch_shapes=[
            pltpu.VMEM((MQ, N_OUT), jnp.float32),
            pltpu.VMEM((MQ, N_OUT), jnp.float32),
            pltpu.SemaphoreType.DMA((C,)),
            pltpu.SemaphoreType.DMA((C,)),
            pltpu.SemaphoreType.DMA((C,)),
            pltpu.SemaphoreType.DMA((C,)),
        ],
        compiler_params=pltpu.CompilerParams(collective_id=0),
    )(x)
